# Initial kernel scaffold; baseline (speedup 1.0000x reference)
import jax
import jax.numpy as jnp
from jax.experimental import pallas as pl


def kernel(dice_state):
    raise NotImplementedError("write your pallas kernel here")


# speedup vs baseline: 8.7591x; 8.7591x over previous
import dataclasses

import jax
import jax.numpy as jnp
from jax import lax
from jax.experimental import pallas as pl
from jax.experimental.pallas import tpu as pltpu
from jax.experimental.pallas import tpu_sc as plsc

L = 16
R = 1024

_CES = ((0, 1), (3, 4), (2, 4), (2, 3), (1, 4), (0, 3), (0, 2), (1, 3), (1, 2))


def _block_body(x_vmem, sorted_vmem, hist_vmem):
    iota = lax.iota(jnp.int32, L)
    iota5 = iota * 5
    iota6 = iota * 6

    @pl.loop(0, R, step=L)
    def _(r0):
        base5 = r0 * 5
        base6 = r0 * 6
        d = [plsc.load_gather(x_vmem, [iota5 + (base5 + j)]) for j in range(5)]

        s = None
        for j in range(5):
            e = d[j].astype(jnp.int32)
            t = jnp.int32(1) << (3 * e - 3)
            s = t if s is None else s + t
        for v in range(6):
            h = (s >> (3 * v)) & 7
            plsc.store_scatter(hist_vmem, [iota6 + (base6 + v)],
                               h.astype(jnp.float32))

        c = list(d)
        for a, b in _CES:
            lo = jnp.minimum(c[a], c[b])
            hi = jnp.maximum(c[a], c[b])
            c[a], c[b] = lo, hi
        for j in range(5):
            plsc.store_scatter(sorted_vmem, [iota5 + (base5 + j)], c[j])


def kernel(dice_state):
    B = dice_state.shape[0]
    x_flat = dice_state.reshape(B * 5)
    mesh = plsc.VectorSubcoreMesh(core_axis_name="c", subcore_axis_name="s")
    cp = pltpu.CompilerParams()
    if "needs_layout_passes" in pltpu.CompilerParams.__dataclass_fields__:
        cp = dataclasses.replace(cp, needs_layout_passes=False)

    @pl.kernel(
        out_type=(
            jax.ShapeDtypeStruct((B * 5,), jnp.float32),
            jax.ShapeDtypeStruct((B * 6,), jnp.float32),
        ),
        mesh=mesh,
        compiler_params=cp,
    )
    def run(x_hbm, sorted_hbm, hist_hbm):
        pltpu.emit_pipeline(
            _block_body,
            grid=(B // R,),
            in_specs=[pl.BlockSpec((R * 5,), lambda i: (i,))],
            out_specs=[
                pl.BlockSpec((R * 5,), lambda i: (i,)),
                pl.BlockSpec((R * 6,), lambda i: (i,)),
            ],
            core_axis_name=("c", "s"),
            dimension_semantics=(pltpu.PARALLEL,),
        )(x_hbm, sorted_hbm, hist_hbm)

    sorted_flat, hist_flat = run(x_flat)
    return sorted_flat.reshape(B, 5), hist_flat.reshape(B, 6)


# speedup vs baseline: 213.1218x; 24.3315x over previous
import dataclasses

import jax
import jax.numpy as jnp
from jax.experimental import pallas as pl
from jax.experimental.pallas import tpu as pltpu
from jax.experimental.pallas import tpu_sc as plsc

L = 16
C = 2048

_CES = ((0, 1), (3, 4), (2, 4), (2, 3), (1, 4), (0, 3), (0, 2), (1, 3), (1, 2))


def _block_body(x_vmem, sorted_vmem, hist_vmem):
    @pl.loop(0, C, step=L)
    def _(c0):
        sl = pl.ds(c0, L)
        d = [x_vmem[j, sl] for j in range(5)]

        s = None
        for j in range(5):
            e = d[j].astype(jnp.int32)
            t = jnp.int32(1) << (3 * e - 3)
            s = t if s is None else s + t
        for v in range(6):
            h = (s >> (3 * v)) & 7
            hist_vmem[v, sl] = h.astype(jnp.float32)

        c = list(d)
        for a, b in _CES:
            lo = jnp.minimum(c[a], c[b])
            hi = jnp.maximum(c[a], c[b])
            c[a], c[b] = lo, hi
        for j in range(5):
            sorted_vmem[j, sl] = c[j]


def kernel(dice_state):
    B = dice_state.shape[0]
    x_t = dice_state.T
    mesh = plsc.VectorSubcoreMesh(core_axis_name="c", subcore_axis_name="s")
    cp = pltpu.CompilerParams()
    fields = pltpu.CompilerParams.__dataclass_fields__
    if "needs_layout_passes" in fields:
        cp = dataclasses.replace(cp, needs_layout_passes=False)
    if "use_tc_tiling_on_sc" in fields:
        cp = dataclasses.replace(cp, use_tc_tiling_on_sc=True)

    @pl.kernel(
        out_type=(
            jax.ShapeDtypeStruct((5, B), jnp.float32),
            jax.ShapeDtypeStruct((6, B), jnp.float32),
        ),
        mesh=mesh,
        compiler_params=cp,
    )
    def run(x_hbm, sorted_hbm, hist_hbm):
        pltpu.emit_pipeline(
            _block_body,
            grid=(B // C,),
            in_specs=[pl.BlockSpec((5, C), lambda i: (0, i))],
            out_specs=[
                pl.BlockSpec((5, C), lambda i: (0, i)),
                pl.BlockSpec((6, C), lambda i: (0, i)),
            ],
            core_axis_name=("c", "s"),
            dimension_semantics=(pltpu.PARALLEL,),
        )(x_hbm, sorted_hbm, hist_hbm)

    sorted_t, hist_t = run(x_t)
    return sorted_t.T, hist_t.T
